# baseline (speedup 1.0000x reference)
import functools

import jax
import jax.numpy as jnp
from jax import lax
from jax.experimental import pallas as pl
from jax.experimental.pallas import tpu as pltpu
from jax.experimental.pallas import tpu_sc as plsc

N = 10000
D = 128
E = 320000

NW = 32
NPT = 320
NPAD = NW * NPT
CH = 8000
NCHUNK = E // CH
GB = 256
TRASH = CH + 16
NEG_INF = float("-inf")

BLK = 1000


def _pre_body(z_ref, w1_ref, w2_ref, b_ref, p_ref, q_ref):
    zb = z_ref[...]
    p_ref[...] = jnp.dot(zb, w1_ref[...], preferred_element_type=jnp.float32)
    q_ref[...] = (jnp.dot(zb, w2_ref[...], preferred_element_type=jnp.float32)
                  + b_ref[...])


_pre_call = pl.pallas_call(
    _pre_body,
    grid=(N // BLK,),
    in_specs=[
        pl.BlockSpec((BLK, D), lambda i: (i, 0)),
        pl.BlockSpec((D, D), lambda i: (0, 0)),
        pl.BlockSpec((D, D), lambda i: (0, 0)),
        pl.BlockSpec((1, D), lambda i: (0, 0)),
    ],
    out_specs=[pl.BlockSpec((BLK, D), lambda i: (i, 0))] * 2,
    out_shape=[jax.ShapeDtypeStruct((N, D), jnp.float32)] * 2,
)


def _post_body(z_ref, g_ref, q_ref, u1_ref, u2_ref, b_ref, o_ref):
    h = g_ref[...] + q_ref[...]
    agg = jnp.where(jnp.isinf(h), 0.0, h)
    o_ref[...] = (jnp.dot(z_ref[...], u1_ref[...], preferred_element_type=jnp.float32)
                  + jnp.dot(agg, u2_ref[...], preferred_element_type=jnp.float32)
                  + b_ref[...])


_post_call = pl.pallas_call(
    _post_body,
    grid=(N // BLK,),
    in_specs=[
        pl.BlockSpec((BLK, D), lambda i: (i, 0)),
        pl.BlockSpec((BLK, D), lambda i: (i, 0)),
        pl.BlockSpec((BLK, D), lambda i: (i, 0)),
        pl.BlockSpec((D, D), lambda i: (0, 0)),
        pl.BlockSpec((D, D), lambda i: (0, 0)),
        pl.BlockSpec((1, D), lambda i: (0, 0)),
    ],
    out_specs=pl.BlockSpec((BLK, D), lambda i: (i, 0)),
    out_shape=jax.ShapeDtypeStruct((N, D), jnp.float32),
)


def _sc_segmax_body(p_hbm, src_hbm, dst_hbm, w_hbm, wcol_hbm, g_hbm,
                    g_loc, dist_buf, src_buf, w_buf, cidx, cdst, cw,
                    rows, wcol_v, sem):
    wid = lax.axis_index("s") * 2 + lax.axis_index("c")
    lo = wid * NPT

    pltpu.sync_copy(wcol_hbm, wcol_v)

    neg = jnp.full((16,), NEG_INF, jnp.float32)

    def init_g(r, carry):
        for c in range(D // 16):
            g_loc[r, pl.ds(c * 16, 16)] = neg
        return carry

    lax.fori_loop(0, NPT + 1, init_g, 0)

    zeros16 = jnp.zeros((16,), jnp.int32)

    def init_c(v, carry):
        cidx[pl.ds(v * 16, 16)] = zeros16
        return carry

    lax.fori_loop(0, (CH + 32) // 16, init_c, 0)

    def chunk_body(ci, carry):
        base = ci * CH
        pltpu.sync_copy(dst_hbm.at[pl.ds(base, CH)], dist_buf)
        pltpu.sync_copy(src_hbm.at[pl.ds(base, CH)], src_buf)
        pltpu.sync_copy(w_hbm.at[pl.ds(base, CH)], w_buf)

        def vec_body(v, cnt):
            sl = pl.ds(v * 16, 16)
            dl = dist_buf[sl] - lo
            m = (dl >= 0) & (dl < NPT)
            cs = jnp.cumsum(m.astype(jnp.int32))
            pos = jnp.where(m, cnt + cs - 1, TRASH)
            plsc.store_scatter(cdst, [pos], dl)
            plsc.store_scatter(cidx, [pos], src_buf[sl])
            plsc.store_scatter(cw, [pos], w_buf[sl])
            return cnt + cs[15]

        k = lax.fori_loop(0, CH // 16, vec_body, jnp.int32(0))

        cdst[pl.ds(k, 16)] = jnp.full((16,), NPT, jnp.int32)

        def batch_body(bi, carry2):
            b = bi * GB
            pltpu.async_copy(p_hbm.at[cidx.at[pl.ds(b, GB)]], rows, sem).wait()
            kk = jnp.minimum(k - b, GB)

            def group_body(gi, carry3):
                jbase = gi * 16
                dvec = cdst[pl.ds(b + jbase, 16)]
                wvec = cw[pl.ds(b + jbase, 16)]
                for jj in range(16):
                    dstl = dvec[jj]
                    we = wvec[jj]
                    for c in range(D // 16):
                        csl = pl.ds(c * 16, 16)
                        mval = rows[jbase + jj, csl] + we * wcol_v[csl]
                        g_loc[dstl, csl] = jnp.maximum(g_loc[dstl, csl], mval)
                return carry3

            lax.fori_loop(0, ((kk + 15) // 16) * 0, group_body, 0)
            return carry2

        nb = (k + GB - 1) // GB
        lax.fori_loop(0, nb, batch_body, 0)
        return carry

    lax.fori_loop(0, NCHUNK, chunk_body, 0)

    pltpu.sync_copy(g_loc.at[pl.ds(0, NPT)], g_hbm.at[pl.ds(lo, NPT)])


_sc_segmax = functools.partial(
    pl.kernel,
    out_type=jax.ShapeDtypeStruct((NPAD, D), jnp.float32),
    mesh=plsc.VectorSubcoreMesh(core_axis_name="c", subcore_axis_name="s"),
    compiler_params=pltpu.CompilerParams(needs_layout_passes=False),
    scratch_types=[
        pltpu.VMEM((NPT + 1, D), jnp.float32),
        pltpu.VMEM((CH,), jnp.int32),
        pltpu.VMEM((CH,), jnp.int32),
        pltpu.VMEM((CH,), jnp.float32),
        pltpu.VMEM((CH + 32,), jnp.int32),
        pltpu.VMEM((CH + 32,), jnp.int32),
        pltpu.VMEM((CH + 32,), jnp.float32),
        pltpu.VMEM((GB, D), jnp.float32),
        pltpu.VMEM((D,), jnp.float32),
        pltpu.SemaphoreType.DMA,
    ],
)(_sc_segmax_body)


def kernel(z, sources, dists, weights, W_msg, b_msg, W_upd, b_upd):
    wmt = W_msg.T
    w1 = wmt[:D]
    w2 = wmt[D:2 * D]
    wcol = wmt[2 * D]
    ut = W_upd.T
    u1 = ut[:D]
    u2 = ut[D:]

    p, q = _pre_call(z, w1, w2, b_msg.reshape(1, D))
    g = _sc_segmax(p,
                   sources.astype(jnp.int32),
                   dists.astype(jnp.int32),
                   weights[:, 0],
                   wcol)
    return _post_call(z, g[:N], q, u1, u2, b_upd.reshape(1, D))


# speedup vs baseline: 1.1894x; 1.1894x over previous
import functools

import jax
import jax.numpy as jnp
from jax import lax
from jax.experimental import pallas as pl
from jax.experimental.pallas import tpu as pltpu
from jax.experimental.pallas import tpu_sc as plsc

N = 10000
D = 128
E = 320000

NPT = 320
NPAD = 32 * NPT
CH = 4000
NCHUNK = E // CH
SB = 32
QD = 6
CPAD = CH + 224
TRASH = CH + 200
NEG_INF = float("-inf")

BLK = 1000


def _pre_body(z_ref, w1_ref, w2_ref, b_ref, p_ref, q_ref):
    zb = z_ref[...]
    p_ref[...] = jnp.dot(zb, w1_ref[...], preferred_element_type=jnp.float32)
    q_ref[...] = (jnp.dot(zb, w2_ref[...], preferred_element_type=jnp.float32)
                  + b_ref[...])


_pre_call = pl.pallas_call(
    _pre_body,
    grid=(N // BLK,),
    in_specs=[
        pl.BlockSpec((BLK, D), lambda i: (i, 0)),
        pl.BlockSpec((D, D), lambda i: (0, 0)),
        pl.BlockSpec((D, D), lambda i: (0, 0)),
        pl.BlockSpec((1, D), lambda i: (0, 0)),
    ],
    out_specs=[pl.BlockSpec((BLK, D), lambda i: (i, 0))] * 2,
    out_shape=[jax.ShapeDtypeStruct((N, D), jnp.float32)] * 2,
)


def _post_body(z_ref, g_ref, q_ref, u1_ref, u2_ref, b_ref, o_ref):
    h = g_ref[...] + q_ref[...]
    agg = jnp.where(jnp.isinf(h), 0.0, h)
    o_ref[...] = (jnp.dot(z_ref[...], u1_ref[...], preferred_element_type=jnp.float32)
                  + jnp.dot(agg, u2_ref[...], preferred_element_type=jnp.float32)
                  + b_ref[...])


_post_call = pl.pallas_call(
    _post_body,
    grid=(N // BLK,),
    in_specs=[
        pl.BlockSpec((BLK, D), lambda i: (i, 0)),
        pl.BlockSpec((BLK, D), lambda i: (i, 0)),
        pl.BlockSpec((BLK, D), lambda i: (i, 0)),
        pl.BlockSpec((D, D), lambda i: (0, 0)),
        pl.BlockSpec((D, D), lambda i: (0, 0)),
        pl.BlockSpec((1, D), lambda i: (0, 0)),
    ],
    out_specs=pl.BlockSpec((BLK, D), lambda i: (i, 0)),
    out_shape=jax.ShapeDtypeStruct((N, D), jnp.float32),
)


def _sc_segmax_body(p_hbm, src_hbm, dst_hbm, w_hbm, wcol_hbm, g_hbm,
                    g_loc, dist_buf, src_buf, w_buf, cidx, cdst, cw,
                    rows, wcol_v, sem):
    core = lax.axis_index("c")
    sid = lax.axis_index("s")
    wid = core * 16 + sid
    lo = wid * NPT

    pltpu.sync_copy(wcol_hbm, wcol_v)

    neg = jnp.full((16,), NEG_INF, jnp.float32)

    def init_g(r, carry):
        g_loc[pl.ds(r * 16, 16)] = neg
        return carry

    lax.fori_loop(0, (NPT + 1) * D // 16, init_g, 0)

    zeros16 = jnp.zeros((16,), jnp.int32)

    def init_c(v, carry):
        cidx[pl.ds(v * 16, 16)] = zeros16
        return carry

    lax.fori_loop(0, CPAD // 16, init_c, 0)

    def chunk_body(ci, carry):
        base = ci * CH
        pltpu.sync_copy(dst_hbm.at[pl.ds(base, CH)], dist_buf)
        pltpu.sync_copy(src_hbm.at[pl.ds(base, CH)], src_buf)
        pltpu.sync_copy(w_hbm.at[pl.ds(base, CH)], w_buf)

        def vec_body(v, cnt):
            sl = pl.ds(v * 16, 16)
            dl = dist_buf[sl] - lo
            m = (dl >= 0) & (dl < NPT)
            cs = jnp.cumsum(m.astype(jnp.int32))
            pos = jnp.where(m, cnt + cs - 1, TRASH)
            plsc.store_scatter(cdst, [pos], dl)
            plsc.store_scatter(cidx, [pos], src_buf[sl])
            plsc.store_scatter(cw, [pos], w_buf[sl])
            return cnt + cs[15]

        k = lax.fori_loop(0, CH // 16, vec_body, jnp.int32(0))

        cdst[pl.ds(k, 16)] = jnp.full((16,), NPT, jnp.int32)

        def round_body(r, carry2):
            rbase = r * (QD * SB)
            descs = []
            for q in range(QD):
                descs.append(pltpu.async_copy(
                    p_hbm.at[cidx.at[pl.ds(rbase + q * SB, SB)]],
                    rows.at[q], sem))
            for q in range(QD):
                descs[q].wait()
            for q in range(QD):
                sbase = rbase + q * SB
                kk = jnp.minimum(k - sbase, SB)

                def group_body(gi, carry3, q=q, sbase=sbase):
                    jbase = gi * 16
                    dvec = cdst[pl.ds(sbase + jbase, 16)]
                    wvec = cw[pl.ds(sbase + jbase, 16)]
                    for jj in range(16):
                        dstl = dvec[jj]
                        we = wvec[jj]
                        gbase = dstl * D
                        for c in range(D // 16):
                            csl = pl.ds(c * 16, 16)
                            gsl = pl.ds(gbase + c * 16, 16)
                            mval = rows[q, jbase + jj, csl] + we * wcol_v[csl]
                            g_loc[gsl] = jnp.maximum(g_loc[gsl], mval)
                    return carry3

                ng = jnp.maximum((kk + 15) // 16, 0)
                lax.fori_loop(0, ng, group_body, 0)
            return carry2

        nsb = (k + SB - 1) // SB
        nrounds = (nsb + QD - 1) // QD
        lax.fori_loop(0, nrounds, round_body, 0)
        return carry

    lax.fori_loop(0, NCHUNK, chunk_body, 0)

    pltpu.sync_copy(g_loc.at[pl.ds(0, NPT * D)], g_hbm.at[wid])


_sc_segmax = functools.partial(
    pl.kernel,
    out_type=jax.ShapeDtypeStruct((32, NPT * D), jnp.float32),
    mesh=plsc.VectorSubcoreMesh(core_axis_name="c", subcore_axis_name="s"),
    compiler_params=pltpu.CompilerParams(needs_layout_passes=False),
    scratch_types=[
        pltpu.VMEM(((NPT + 1) * D,), jnp.float32),
        pltpu.VMEM((CH,), jnp.int32),
        pltpu.VMEM((CH,), jnp.int32),
        pltpu.VMEM((CH,), jnp.float32),
        pltpu.VMEM((CPAD,), jnp.int32),
        pltpu.VMEM((CPAD,), jnp.int32),
        pltpu.VMEM((CPAD,), jnp.float32),
        pltpu.VMEM((QD, SB, D), jnp.float32),
        pltpu.VMEM((D,), jnp.float32),
        pltpu.SemaphoreType.DMA,
    ],
)(_sc_segmax_body)


def kernel(z, sources, dists, weights, W_msg, b_msg, W_upd, b_upd):
    wmt = W_msg.T
    w1 = wmt[:D]
    w2 = wmt[D:2 * D]
    wcol = wmt[2 * D]
    ut = W_upd.T
    u1 = ut[:D]
    u2 = ut[D:]

    p, q = _pre_call(z, w1, w2, b_msg.reshape(1, D))
    g3 = _sc_segmax(p,
                    sources.astype(jnp.int32),
                    dists.astype(jnp.int32),
                    weights[:, 0],
                    wcol)
    g = g3.reshape(NPAD, D)[:N]
    return _post_call(z, g, q, u1, u2, b_upd.reshape(1, D))
